# baseline (speedup 1.0000x reference)
import functools

import jax
import jax.numpy as jnp
from jax import lax
from jax.experimental import pallas as pl
from jax.experimental.pallas import tpu as pltpu
from jax.experimental.pallas import tpu_sc as plsc

B = 16384
F = 26
V = 100000
D = 18
FH = F // 2
SLABS_H = FH * D
NW = 32
CHUNK = 2048


@functools.cache
def _build_sc_gather(f_base):
    mesh = plsc.VectorSubcoreMesh(core_axis_name="c", subcore_axis_name="s")

    @functools.partial(
        pl.kernel,
        mesh=mesh,
        compiler_params=pltpu.CompilerParams(needs_layout_passes=False),
        out_type=jax.ShapeDtypeStruct((SLABS_H * B,), jnp.float32),
        scratch_types=[
            pltpu.VMEM((V,), jnp.float32),
            pltpu.VMEM((B,), jnp.int32),
            pltpu.VMEM((CHUNK,), jnp.float32),
        ],
    )
    def _sc_gather(tab_hbm, idx_hbm, out_hbm, slab_v, idx_v, out_v):
        w = lax.axis_index("s") * 2 + lax.axis_index("c")
        lo = 7 * w + jnp.minimum(w, 10)
        hi = lo + 7 + (w < 10).astype(jnp.int32)

        def field_body(f, _):
            s0 = f * D

            @pl.when(jnp.logical_and(s0 < hi, s0 + D > lo))
            def _():
                pltpu.sync_copy(idx_hbm.at[pl.ds((f_base + f) * B, B)], idx_v)

                def d_body(d, _):
                    s = s0 + d

                    @pl.when(jnp.logical_and(s >= lo, s < hi))
                    def _():
                        pltpu.sync_copy(tab_hbm.at[d, f_base + f], slab_v)

                        def chunk_body(c, _):
                            @plsc.parallel_loop(0, CHUNK, step=16, unroll=8)
                            def _g(o):
                                iv = idx_v[pl.ds(c * CHUNK + o, 16)]
                                out_v[pl.ds(o, 16)] = plsc.load_gather(
                                    slab_v, [iv])

                            pltpu.sync_copy(
                                out_v,
                                out_hbm.at[pl.ds(s * B + c * CHUNK, CHUNK)])
                            return 0

                        lax.fori_loop(0, B // CHUNK, chunk_body, 0)

                    return 0

                lax.fori_loop(0, D, d_body, 0)

            return 0

        lax.fori_loop(0, FH, field_body, 0)

    return _sc_gather


M = 16


def _mlp_part_body(x_ref, w1_ref, o_ref):
    x = x_ref[...].reshape(SLABS_H, M * 128)
    o_ref[...] = lax.dot_general(x, w1_ref[...], (((0,), (0,)), ((), ())),
                                 preferred_element_type=jnp.float32)


_mlp_part = pl.pallas_call(
    _mlp_part_body,
    grid=(128 // M,),
    in_specs=[
        pl.BlockSpec((SLABS_H, M, 128), lambda i: (0, i, 0)),
        pl.BlockSpec((SLABS_H, 64), lambda i: (0, 0)),
    ],
    out_specs=pl.BlockSpec((M * 128, 64), lambda i: (i, 0)),
    out_shape=jax.ShapeDtypeStruct((B, 64), jnp.float32),
)


def _mlp_fin_body(x_ref, h1a_ref, num_ref, w1_ref, w1n_ref, b1_ref, w2_ref,
                  b2_ref, w3_ref, b3_ref, o_ref):
    x = x_ref[...].reshape(SLABS_H, M * 128)
    h = lax.dot_general(x, w1_ref[...], (((0,), (0,)), ((), ())),
                        preferred_element_type=jnp.float32)
    h = h + h1a_ref[...] + num_ref[...] * w1n_ref[...] + b1_ref[...]
    h = jnp.maximum(h, 0.0)
    h = jnp.dot(h, w2_ref[...], preferred_element_type=jnp.float32)
    h = jnp.maximum(h + b2_ref[...], 0.0)
    o_ref[...] = (jnp.dot(h, w3_ref[...], preferred_element_type=jnp.float32)
                  + b3_ref[...])


_mlp_fin = pl.pallas_call(
    _mlp_fin_body,
    grid=(128 // M,),
    in_specs=[
        pl.BlockSpec((SLABS_H, M, 128), lambda i: (0, i, 0)),
        pl.BlockSpec((M * 128, 64), lambda i: (i, 0)),
        pl.BlockSpec((M * 128, 1), lambda i: (i, 0)),
        pl.BlockSpec((SLABS_H, 64), lambda i: (0, 0)),
        pl.BlockSpec((1, 64), lambda i: (0, 0)),
        pl.BlockSpec((1, 64), lambda i: (0, 0)),
        pl.BlockSpec((64, 32), lambda i: (0, 0)),
        pl.BlockSpec((1, 32), lambda i: (0, 0)),
        pl.BlockSpec((32, 3), lambda i: (0, 0)),
        pl.BlockSpec((1, 3), lambda i: (0, 0)),
    ],
    out_specs=pl.BlockSpec((M * 128, 3), lambda i: (i, 0)),
    out_shape=jax.ShapeDtypeStruct((B, 3), jnp.float32),
)


def kernel(numeric, cat_indices, tables, W1, b1, W2, b2, W3, b3):
    tabT = tables.transpose(2, 0, 1)
    idx_fmaj = cat_indices.astype(jnp.int32).T.reshape(-1)
    flat_a = _build_sc_gather(0)(tabT, idx_fmaj)
    flat_b = _build_sc_gather(FH)(tabT, idx_fmaj)
    x3a = flat_a.reshape(SLABS_H, 128, 128)
    x3b = flat_b.reshape(SLABS_H, 128, 128)
    w1e = W1[1:, :]
    h1a = _mlp_part(x3a, w1e[:SLABS_H])
    return _mlp_fin(x3b, h1a, numeric, w1e[SLABS_H:], W1[0:1, :],
                    b1[None, :], W2, b2[None, :], W3, b3[None, :])


# speedup vs baseline: 1.0571x; 1.0571x over previous
import functools

import jax
import jax.numpy as jnp
from jax import lax
from jax.experimental import pallas as pl
from jax.experimental.pallas import tpu as pltpu
from jax.experimental.pallas import tpu_sc as plsc

B = 16384
F = 26
V = 100000
D = 18
SLABS = F * D
NW = 32
CHUNK = 2048


@functools.cache
def _build_sc_gather():
    mesh = plsc.VectorSubcoreMesh(core_axis_name="c", subcore_axis_name="s")

    @functools.partial(
        pl.kernel,
        mesh=mesh,
        compiler_params=pltpu.CompilerParams(needs_layout_passes=False),
        out_type=jax.ShapeDtypeStruct((SLABS * B,), jnp.float32),
        scratch_types=[
            pltpu.VMEM((V,), jnp.float32),
            pltpu.VMEM((B,), jnp.int32),
            pltpu.VMEM((CHUNK,), jnp.float32),
        ],
    )
    def _sc_gather(tab_hbm, idx_hbm, out_hbm, slab_v, idx_v, out_v):
        w = lax.axis_index("s") * 2 + lax.axis_index("c")
        lo = 14 * w + jnp.minimum(w, 20)
        hi = lo + 14 + (w < 20).astype(jnp.int32)

        def field_body(f, _):
            s0 = f * D

            @pl.when(jnp.logical_and(s0 < hi, s0 + D > lo))
            def _():
                pltpu.sync_copy(idx_hbm.at[pl.ds(f * B, B)], idx_v)

                def d_body(d, _):
                    s = s0 + d

                    @pl.when(jnp.logical_and(s >= lo, s < hi))
                    def _():
                        pltpu.sync_copy(tab_hbm.at[d, f], slab_v)

                        def chunk_body(c, _):
                            @plsc.parallel_loop(0, CHUNK, step=16, unroll=8)
                            def _g(o):
                                iv = idx_v[pl.ds(c * CHUNK + o, 16)]
                                out_v[pl.ds(o, 16)] = plsc.load_gather(
                                    slab_v, [iv])

                            pltpu.sync_copy(
                                out_v,
                                out_hbm.at[pl.ds(s * B + c * CHUNK, CHUNK)])
                            return 0

                        lax.fori_loop(0, B // CHUNK, chunk_body, 0)

                    return 0

                lax.fori_loop(0, D, d_body, 0)

            return 0

        lax.fori_loop(0, F, field_body, 0)

    return _sc_gather


M = 8


def _mlp_body(x_ref, num_ref, w1_ref, w1n_ref, b1_ref, w2_ref, b2_ref,
              w3_ref, b3_ref, o_ref):
    x = x_ref[...].reshape(SLABS, M * 128)
    h = lax.dot_general(x, w1_ref[...], (((0,), (0,)), ((), ())),
                        preferred_element_type=jnp.float32)
    h = jnp.maximum(h + num_ref[...] * w1n_ref[...] + b1_ref[...], 0.0)
    h = jnp.dot(h, w2_ref[...], preferred_element_type=jnp.float32)
    h = jnp.maximum(h + b2_ref[...], 0.0)
    o_ref[...] = (jnp.dot(h, w3_ref[...], preferred_element_type=jnp.float32)
                  + b3_ref[...])


_mlp_call = pl.pallas_call(
    _mlp_body,
    grid=(128 // M,),
    in_specs=[
        pl.BlockSpec((SLABS, M, 128), lambda i: (0, i, 0)),
        pl.BlockSpec((M * 128, 1), lambda i: (i, 0)),
        pl.BlockSpec((SLABS, 64), lambda i: (0, 0)),
        pl.BlockSpec((1, 64), lambda i: (0, 0)),
        pl.BlockSpec((1, 64), lambda i: (0, 0)),
        pl.BlockSpec((64, 32), lambda i: (0, 0)),
        pl.BlockSpec((1, 32), lambda i: (0, 0)),
        pl.BlockSpec((32, 3), lambda i: (0, 0)),
        pl.BlockSpec((1, 3), lambda i: (0, 0)),
    ],
    out_specs=pl.BlockSpec((M * 128, 3), lambda i: (i, 0)),
    out_shape=jax.ShapeDtypeStruct((B, 3), jnp.float32),
)


def kernel(numeric, cat_indices, tables, W1, b1, W2, b2, W3, b3):
    tabT = tables.transpose(2, 0, 1)
    idx_fmaj = cat_indices.astype(jnp.int32).T.reshape(-1)
    flat = _build_sc_gather()(tabT, idx_fmaj)
    x3 = flat.reshape(SLABS, 128, 128)
    return _mlp_call(x3, numeric, W1[1:, :], W1[0:1, :], b1[None, :],
                     W2, b2[None, :], W3, b3[None, :])


# speedup vs baseline: 1.0773x; 1.0191x over previous
import functools

import jax
import jax.numpy as jnp
from jax import lax
from jax.experimental import pallas as pl
from jax.experimental.pallas import tpu as pltpu
from jax.experimental.pallas import tpu_sc as plsc

B = 16384
F = 26
V = 100000
D = 18
SLABS = F * D
NW = 32
CHUNK = 2048


@functools.cache
def _build_sc_gather():
    mesh = plsc.VectorSubcoreMesh(core_axis_name="c", subcore_axis_name="s")

    @functools.partial(
        pl.kernel,
        mesh=mesh,
        compiler_params=pltpu.CompilerParams(needs_layout_passes=False),
        out_type=jax.ShapeDtypeStruct((SLABS * B,), jnp.float32),
        scratch_types=[
            pltpu.VMEM((V,), jnp.float32),
            pltpu.VMEM((B,), jnp.int32),
            pltpu.VMEM((CHUNK,), jnp.float32),
        ],
    )
    def _sc_gather(tab_hbm, idx_hbm, out_hbm, slab_v, idx_v, out_v):
        w = lax.axis_index("s") * 2 + lax.axis_index("c")
        lo = 14 * w + jnp.minimum(w, 20)
        hi = lo + 14 + (w < 20).astype(jnp.int32)

        def field_body(f, _):
            s0 = f * D

            @pl.when(jnp.logical_and(s0 < hi, s0 + D > lo))
            def _():
                pltpu.sync_copy(idx_hbm.at[pl.ds(f * B, B)], idx_v)

                def d_body(d, _):
                    s = s0 + d

                    @pl.when(jnp.logical_and(s >= lo, s < hi))
                    def _():
                        pltpu.sync_copy(tab_hbm.at[d, f], slab_v)

                        def chunk_body(c, _):
                            @plsc.parallel_loop(0, CHUNK, step=16, unroll=8)
                            def _g(o):
                                iv = idx_v[pl.ds(c * CHUNK + o, 16)]
                                out_v[pl.ds(o, 16)] = plsc.load_gather(
                                    slab_v, [iv])

                            pltpu.sync_copy(
                                out_v,
                                out_hbm.at[pl.ds(s * B + c * CHUNK, CHUNK)])
                            return 0

                        lax.fori_loop(0, B // CHUNK, chunk_body, 0)

                    return 0

                lax.fori_loop(0, D, d_body, 0)

            return 0

        lax.fori_loop(0, F, field_body, 0)

    return _sc_gather


M = 32


def _mlp_body(x_ref, num_ref, w1_ref, w1n_ref, b1_ref, w2_ref, b2_ref,
              w3_ref, b3_ref, o_ref):
    x = x_ref[...].reshape(SLABS, M * 128)
    h = lax.dot_general(x, w1_ref[...], (((0,), (0,)), ((), ())),
                        preferred_element_type=jnp.float32)
    h = jnp.maximum(h + num_ref[...] * w1n_ref[...] + b1_ref[...], 0.0)
    h = jnp.dot(h, w2_ref[...], preferred_element_type=jnp.float32)
    h = jnp.maximum(h + b2_ref[...], 0.0)
    o_ref[...] = (jnp.dot(h, w3_ref[...], preferred_element_type=jnp.float32)
                  + b3_ref[...])


_mlp_call = pl.pallas_call(
    _mlp_body,
    grid=(128 // M,),
    in_specs=[
        pl.BlockSpec((SLABS, M, 128), lambda i: (0, i, 0)),
        pl.BlockSpec((M * 128, 1), lambda i: (i, 0)),
        pl.BlockSpec((SLABS, 64), lambda i: (0, 0)),
        pl.BlockSpec((1, 64), lambda i: (0, 0)),
        pl.BlockSpec((1, 64), lambda i: (0, 0)),
        pl.BlockSpec((64, 32), lambda i: (0, 0)),
        pl.BlockSpec((1, 32), lambda i: (0, 0)),
        pl.BlockSpec((32, 3), lambda i: (0, 0)),
        pl.BlockSpec((1, 3), lambda i: (0, 0)),
    ],
    out_specs=pl.BlockSpec((M * 128, 3), lambda i: (i, 0)),
    out_shape=jax.ShapeDtypeStruct((B, 3), jnp.float32),
)


def kernel(numeric, cat_indices, tables, W1, b1, W2, b2, W3, b3):
    tabT = tables.transpose(2, 0, 1)
    idx_fmaj = cat_indices.astype(jnp.int32).T.reshape(-1)
    flat = _build_sc_gather()(tabT, idx_fmaj)
    x3 = flat.reshape(SLABS, 128, 128)
    return _mlp_call(x3, numeric, W1[1:, :], W1[0:1, :], b1[None, :],
                     W2, b2[None, :], W3, b3[None, :])


# speedup vs baseline: 1.1589x; 1.0758x over previous
import functools

import jax
import jax.numpy as jnp
from jax import lax
from jax.experimental import pallas as pl
from jax.experimental.pallas import tpu as pltpu
from jax.experimental.pallas import tpu_sc as plsc

B = 16384
F = 26
V = 100000
D = 18
SLABS = F * D
ROWS = SLABS + 1
NW = 32
CHUNK = 2048


@functools.cache
def _build_sc_gather():
    mesh = plsc.VectorSubcoreMesh(core_axis_name="c", subcore_axis_name="s")

    @functools.partial(
        pl.kernel,
        mesh=mesh,
        compiler_params=pltpu.CompilerParams(needs_layout_passes=False),
        out_type=jax.ShapeDtypeStruct((ROWS * B,), jnp.float32),
        scratch_types=[
            pltpu.VMEM((V,), jnp.float32),
            pltpu.VMEM((B,), jnp.int32),
            pltpu.VMEM((CHUNK,), jnp.float32),
        ],
    )
    def _sc_gather(tab_hbm, idx_hbm, num_hbm, out_hbm, slab_v, idx_v, out_v):
        w = lax.axis_index("s") * 2 + lax.axis_index("c")
        lo = 14 * w + jnp.minimum(w, 20)
        hi = lo + 14 + (w < 20).astype(jnp.int32)

        @pl.when(w == 31)
        def _():
            pltpu.sync_copy(num_hbm, out_hbm.at[pl.ds(SLABS * B, B)])

        def field_body(f, _):
            s0 = f * D

            @pl.when(jnp.logical_and(s0 < hi, s0 + D > lo))
            def _():
                pltpu.sync_copy(idx_hbm.at[pl.ds(f * B, B)], idx_v)

                def d_body(d, _):
                    s = s0 + d

                    @pl.when(jnp.logical_and(s >= lo, s < hi))
                    def _():
                        pltpu.sync_copy(tab_hbm.at[d, f], slab_v)

                        def chunk_body(c, _):
                            @plsc.parallel_loop(0, CHUNK, step=16, unroll=8)
                            def _g(o):
                                iv = idx_v[pl.ds(c * CHUNK + o, 16)]
                                out_v[pl.ds(o, 16)] = plsc.load_gather(
                                    slab_v, [iv])

                            pltpu.sync_copy(
                                out_v,
                                out_hbm.at[pl.ds(s * B + c * CHUNK, CHUNK)])
                            return 0

                        lax.fori_loop(0, B // CHUNK, chunk_body, 0)

                    return 0

                lax.fori_loop(0, D, d_body, 0)

            return 0

        lax.fori_loop(0, F, field_body, 0)

    return _sc_gather


M = 16


def _mlp_body(x_ref, w1_ref, b1_ref, w2_ref, b2_ref, w3_ref, b3t_ref, o_ref):
    x = x_ref[...].reshape(ROWS, M * 128)
    h = lax.dot_general(x, w1_ref[...], (((0,), (0,)), ((), ())),
                        preferred_element_type=jnp.float32)
    h = jnp.maximum(h + b1_ref[...], 0.0)
    h = jnp.dot(h, w2_ref[...], preferred_element_type=jnp.float32)
    h = jnp.maximum(h + b2_ref[...], 0.0)
    o_ref[...] = (lax.dot_general(w3_ref[...], h, (((0,), (1,)), ((), ())),
                                  preferred_element_type=jnp.float32)
                  + b3t_ref[...])


_mlp_call = pl.pallas_call(
    _mlp_body,
    grid=(128 // M,),
    in_specs=[
        pl.BlockSpec((ROWS, M, 128), lambda i: (0, i, 0)),
        pl.BlockSpec((ROWS, 64), lambda i: (0, 0)),
        pl.BlockSpec((1, 64), lambda i: (0, 0)),
        pl.BlockSpec((64, 32), lambda i: (0, 0)),
        pl.BlockSpec((1, 32), lambda i: (0, 0)),
        pl.BlockSpec((32, 3), lambda i: (0, 0)),
        pl.BlockSpec((3, 1), lambda i: (0, 0)),
    ],
    out_specs=pl.BlockSpec((3, M * 128), lambda i: (0, i)),
    out_shape=jax.ShapeDtypeStruct((3, B), jnp.float32),
)


def kernel(numeric, cat_indices, tables, W1, b1, W2, b2, W3, b3):
    tabT = tables.transpose(2, 0, 1)
    idx_fmaj = cat_indices.astype(jnp.int32).T.reshape(-1)
    num1d = numeric.reshape(B)
    flat = _build_sc_gather()(tabT, idx_fmaj, num1d)
    x3 = flat.reshape(ROWS, 128, 128)
    w1x = jnp.concatenate([W1[1:, :], W1[0:1, :]], axis=0)
    out_t = _mlp_call(x3, w1x, b1[None, :], W2, b2[None, :], W3, b3[:, None])
    return out_t.T


# speedup vs baseline: 1.1908x; 1.0275x over previous
import functools

import jax
import jax.numpy as jnp
from jax import lax
from jax.experimental import pallas as pl
from jax.experimental.pallas import tpu as pltpu
from jax.experimental.pallas import tpu_sc as plsc

B = 16384
F = 26
V = 100000
D = 18
SLABS = F * D
ROWS = SLABS + 1
NW = 32
CHUNK = 4096


@functools.cache
def _build_sc_gather():
    mesh = plsc.VectorSubcoreMesh(core_axis_name="c", subcore_axis_name="s")

    @functools.partial(
        pl.kernel,
        mesh=mesh,
        compiler_params=pltpu.CompilerParams(needs_layout_passes=False),
        out_type=jax.ShapeDtypeStruct((ROWS * B,), jnp.float32),
        scratch_types=[
            pltpu.VMEM((V,), jnp.float32),
            pltpu.VMEM((B,), jnp.int32),
            pltpu.VMEM((CHUNK,), jnp.float32),
        ],
    )
    def _sc_gather(tab_hbm, idx_hbm, num_hbm, out_hbm, slab_v, idx_v, out_v):
        w = lax.axis_index("s") * 2 + lax.axis_index("c")
        lo = 14 * w + jnp.minimum(w, 20)
        hi = lo + 14 + (w < 20).astype(jnp.int32)

        @pl.when(w == 31)
        def _():
            pltpu.sync_copy(num_hbm, out_hbm.at[pl.ds(SLABS * B, B)])

        def field_body(f, _):
            s0 = f * D

            @pl.when(jnp.logical_and(s0 < hi, s0 + D > lo))
            def _():
                pltpu.sync_copy(idx_hbm.at[pl.ds(f * B, B)], idx_v)

                def d_body(d, _):
                    s = s0 + d

                    @pl.when(jnp.logical_and(s >= lo, s < hi))
                    def _():
                        pltpu.sync_copy(tab_hbm.at[d, f], slab_v)

                        def chunk_body(c, _):
                            @plsc.parallel_loop(0, CHUNK, step=16, unroll=8)
                            def _g(o):
                                iv = idx_v[pl.ds(c * CHUNK + o, 16)]
                                out_v[pl.ds(o, 16)] = plsc.load_gather(
                                    slab_v, [iv])

                            pltpu.sync_copy(
                                out_v,
                                out_hbm.at[pl.ds(s * B + c * CHUNK, CHUNK)])
                            return 0

                        lax.fori_loop(0, B // CHUNK, chunk_body, 0)

                    return 0

                lax.fori_loop(0, D, d_body, 0)

            return 0

        lax.fori_loop(0, F, field_body, 0)

    return _sc_gather


M = 16


def _mlp_body(x_ref, w1_ref, b1_ref, w2_ref, b2_ref, w3_ref, b3t_ref, o_ref):
    x = x_ref[...].reshape(ROWS, M * 128)
    h = lax.dot_general(x, w1_ref[...], (((0,), (0,)), ((), ())),
                        preferred_element_type=jnp.float32)
    h = jnp.maximum(h + b1_ref[...], 0.0)
    h = jnp.dot(h, w2_ref[...], preferred_element_type=jnp.float32)
    h = jnp.maximum(h + b2_ref[...], 0.0)
    o_ref[...] = (lax.dot_general(w3_ref[...], h, (((0,), (1,)), ((), ())),
                                  preferred_element_type=jnp.float32)
                  + b3t_ref[...])


_mlp_call = pl.pallas_call(
    _mlp_body,
    grid=(128 // M,),
    in_specs=[
        pl.BlockSpec((ROWS, M, 128), lambda i: (0, i, 0)),
        pl.BlockSpec((ROWS, 64), lambda i: (0, 0)),
        pl.BlockSpec((1, 64), lambda i: (0, 0)),
        pl.BlockSpec((64, 32), lambda i: (0, 0)),
        pl.BlockSpec((1, 32), lambda i: (0, 0)),
        pl.BlockSpec((32, 3), lambda i: (0, 0)),
        pl.BlockSpec((3, 1), lambda i: (0, 0)),
    ],
    out_specs=pl.BlockSpec((3, M * 128), lambda i: (0, i)),
    out_shape=jax.ShapeDtypeStruct((3, B), jnp.float32),
)


def kernel(numeric, cat_indices, tables, W1, b1, W2, b2, W3, b3):
    tabT = tables.transpose(2, 0, 1)
    idx_fmaj = cat_indices.astype(jnp.int32).T.reshape(-1)
    num1d = numeric.reshape(B)
    flat = _build_sc_gather()(tabT, idx_fmaj, num1d)
    x3 = flat.reshape(ROWS, 128, 128)
    w1x = jnp.concatenate([W1[1:, :], W1[0:1, :]], axis=0)
    out_t = _mlp_call(x3, w1x, b1[None, :], W2, b2[None, :], W3, b3[:, None])
    return out_t.T


# speedup vs baseline: 1.2024x; 1.0097x over previous
import functools

import jax
import jax.numpy as jnp
from jax import lax
from jax.experimental import pallas as pl
from jax.experimental.pallas import tpu as pltpu
from jax.experimental.pallas import tpu_sc as plsc

B = 16384
F = 26
V = 100000
D = 18
SLABS = F * D
ROWS = SLABS + 1
NW = 32
CHUNK = 8192


@functools.cache
def _build_sc_gather():
    mesh = plsc.VectorSubcoreMesh(core_axis_name="c", subcore_axis_name="s")

    @functools.partial(
        pl.kernel,
        mesh=mesh,
        compiler_params=pltpu.CompilerParams(needs_layout_passes=False),
        out_type=jax.ShapeDtypeStruct((ROWS * B,), jnp.float32),
        scratch_types=[
            pltpu.VMEM((V,), jnp.float32),
            pltpu.VMEM((B,), jnp.int32),
            pltpu.VMEM((CHUNK,), jnp.float32),
        ],
    )
    def _sc_gather(tab_hbm, idx_hbm, num_hbm, out_hbm, slab_v, idx_v, out_v):
        w = lax.axis_index("s") * 2 + lax.axis_index("c")
        lo = 14 * w + jnp.minimum(w, 20)
        hi = lo + 14 + (w < 20).astype(jnp.int32)

        @pl.when(w == 31)
        def _():
            pltpu.sync_copy(num_hbm, out_hbm.at[pl.ds(SLABS * B, B)])

        def field_body(f, _):
            s0 = f * D

            @pl.when(jnp.logical_and(s0 < hi, s0 + D > lo))
            def _():
                pltpu.sync_copy(idx_hbm.at[pl.ds(f * B, B)], idx_v)

                def d_body(d, _):
                    s = s0 + d

                    @pl.when(jnp.logical_and(s >= lo, s < hi))
                    def _():
                        pltpu.sync_copy(tab_hbm.at[d, f], slab_v)

                        def chunk_body(c, _):
                            @plsc.parallel_loop(0, CHUNK, step=16, unroll=8)
                            def _g(o):
                                iv = idx_v[pl.ds(c * CHUNK + o, 16)]
                                out_v[pl.ds(o, 16)] = plsc.load_gather(
                                    slab_v, [iv])

                            pltpu.sync_copy(
                                out_v,
                                out_hbm.at[pl.ds(s * B + c * CHUNK, CHUNK)])
                            return 0

                        lax.fori_loop(0, B // CHUNK, chunk_body, 0)

                    return 0

                lax.fori_loop(0, D, d_body, 0)

            return 0

        lax.fori_loop(0, F, field_body, 0)

    return _sc_gather


M = 16


def _mlp_body(x_ref, w1_ref, b1_ref, w2_ref, b2_ref, w3_ref, b3t_ref, o_ref):
    x = x_ref[...].reshape(ROWS, M * 128)
    h = lax.dot_general(x, w1_ref[...], (((0,), (0,)), ((), ())),
                        preferred_element_type=jnp.float32)
    h = jnp.maximum(h + b1_ref[...], 0.0)
    h = jnp.dot(h, w2_ref[...], preferred_element_type=jnp.float32)
    h = jnp.maximum(h + b2_ref[...], 0.0)
    o_ref[...] = (lax.dot_general(w3_ref[...], h, (((0,), (1,)), ((), ())),
                                  preferred_element_type=jnp.float32)
                  + b3t_ref[...])


_mlp_call = pl.pallas_call(
    _mlp_body,
    grid=(128 // M,),
    in_specs=[
        pl.BlockSpec((ROWS, M, 128), lambda i: (0, i, 0)),
        pl.BlockSpec((ROWS, 64), lambda i: (0, 0)),
        pl.BlockSpec((1, 64), lambda i: (0, 0)),
        pl.BlockSpec((64, 32), lambda i: (0, 0)),
        pl.BlockSpec((1, 32), lambda i: (0, 0)),
        pl.BlockSpec((32, 3), lambda i: (0, 0)),
        pl.BlockSpec((3, 1), lambda i: (0, 0)),
    ],
    out_specs=pl.BlockSpec((3, M * 128), lambda i: (0, i)),
    out_shape=jax.ShapeDtypeStruct((3, B), jnp.float32),
)


def kernel(numeric, cat_indices, tables, W1, b1, W2, b2, W3, b3):
    tabT = tables.transpose(2, 0, 1)
    idx_fmaj = cat_indices.astype(jnp.int32).T.reshape(-1)
    num1d = numeric.reshape(B)
    flat = _build_sc_gather()(tabT, idx_fmaj, num1d)
    x3 = flat.reshape(ROWS, 128, 128)
    w1x = jnp.concatenate([W1[1:, :], W1[0:1, :]], axis=0)
    out_t = _mlp_call(x3, w1x, b1[None, :], W2, b2[None, :], W3, b3[:, None])
    return out_t.T


# speedup vs baseline: 1.2038x; 1.0011x over previous
import functools

import jax
import jax.numpy as jnp
from jax import lax
from jax.experimental import pallas as pl
from jax.experimental.pallas import tpu as pltpu
from jax.experimental.pallas import tpu_sc as plsc

B = 16384
F = 26
V = 100000
D = 18
SLABS = F * D
ROWS = SLABS + 1
NW = 32
CHUNK = 8192


@functools.cache
def _build_sc_gather():
    mesh = plsc.VectorSubcoreMesh(core_axis_name="c", subcore_axis_name="s")

    @functools.partial(
        pl.kernel,
        mesh=mesh,
        compiler_params=pltpu.CompilerParams(needs_layout_passes=False),
        out_type=jax.ShapeDtypeStruct((ROWS * B,), jnp.float32),
        scratch_types=[
            pltpu.VMEM((V,), jnp.float32),
            pltpu.VMEM((B,), jnp.int32),
            pltpu.VMEM((CHUNK,), jnp.float32),
        ],
    )
    def _sc_gather(tab_hbm, idx_hbm, num_hbm, out_hbm, slab_v, idx_v, out_v):
        w = lax.axis_index("s") * 2 + lax.axis_index("c")
        lo = 14 * w + jnp.minimum(w, 20)
        hi = lo + 14 + (w < 20).astype(jnp.int32)

        @pl.when(w == 31)
        def _():
            pltpu.sync_copy(num_hbm, out_hbm.at[pl.ds(SLABS * B, B)])

        def field_body(f, _):
            s0 = f * D

            @pl.when(jnp.logical_and(s0 < hi, s0 + D > lo))
            def _():
                pltpu.sync_copy(idx_hbm.at[pl.ds(f * B, B)], idx_v)

                def d_body(d, _):
                    s = s0 + d

                    @pl.when(jnp.logical_and(s >= lo, s < hi))
                    def _():
                        pltpu.sync_copy(tab_hbm.at[d, f], slab_v)

                        def chunk_body(c, _):
                            @plsc.parallel_loop(0, CHUNK, step=16, unroll=16)
                            def _g(o):
                                iv = idx_v[pl.ds(c * CHUNK + o, 16)]
                                out_v[pl.ds(o, 16)] = plsc.load_gather(
                                    slab_v, [iv])

                            pltpu.sync_copy(
                                out_v,
                                out_hbm.at[pl.ds(s * B + c * CHUNK, CHUNK)])
                            return 0

                        lax.fori_loop(0, B // CHUNK, chunk_body, 0)

                    return 0

                lax.fori_loop(0, D, d_body, 0)

            return 0

        lax.fori_loop(0, F, field_body, 0)

    return _sc_gather


M = 16


def _mlp_body(x_ref, w1_ref, b1_ref, w2_ref, b2_ref, w3_ref, b3t_ref, o_ref):
    x = x_ref[...].reshape(ROWS, M * 128)
    h = lax.dot_general(x, w1_ref[...], (((0,), (0,)), ((), ())),
                        preferred_element_type=jnp.float32)
    h = jnp.maximum(h + b1_ref[...], 0.0)
    h = jnp.dot(h, w2_ref[...], preferred_element_type=jnp.float32)
    h = jnp.maximum(h + b2_ref[...], 0.0)
    o_ref[...] = (lax.dot_general(w3_ref[...], h, (((0,), (1,)), ((), ())),
                                  preferred_element_type=jnp.float32)
                  + b3t_ref[...])


_mlp_call = pl.pallas_call(
    _mlp_body,
    grid=(128 // M,),
    in_specs=[
        pl.BlockSpec((ROWS, M, 128), lambda i: (0, i, 0)),
        pl.BlockSpec((ROWS, 64), lambda i: (0, 0)),
        pl.BlockSpec((1, 64), lambda i: (0, 0)),
        pl.BlockSpec((64, 32), lambda i: (0, 0)),
        pl.BlockSpec((1, 32), lambda i: (0, 0)),
        pl.BlockSpec((32, 3), lambda i: (0, 0)),
        pl.BlockSpec((3, 1), lambda i: (0, 0)),
    ],
    out_specs=pl.BlockSpec((3, M * 128), lambda i: (0, i)),
    out_shape=jax.ShapeDtypeStruct((3, B), jnp.float32),
)


def kernel(numeric, cat_indices, tables, W1, b1, W2, b2, W3, b3):
    tabT = tables.transpose(2, 0, 1)
    idx_fmaj = cat_indices.astype(jnp.int32).T.reshape(-1)
    num1d = numeric.reshape(B)
    flat = _build_sc_gather()(tabT, idx_fmaj, num1d)
    x3 = flat.reshape(ROWS, 128, 128)
    w1x = jnp.concatenate([W1[1:, :], W1[0:1, :]], axis=0)
    out_t = _mlp_call(x3, w1x, b1[None, :], W2, b2[None, :], W3, b3[:, None])
    return out_t.T
